# baseline (speedup 1.0000x reference)
import functools

import jax
import jax.numpy as jnp
import numpy as np
from jax import lax
from jax.experimental import pallas as pl
from jax.experimental.pallas import tpu as pltpu
from jax.experimental.pallas import tpu_sc as plsc

B = 16384
EMB_K = 16
NROWS = 1000000
ROWS_PER_128 = 8
TBL_ROWS = NROWS // ROWS_PER_128
NC = 2
NS = 16
NW = NC * NS
BPW = B // NW
CHUNK = 128
NCHUNK = BPW // CHUNK
LANE = 16

TCH = 512
NFULL = NROWS // TCH
CPW = NFULL // NW
QCH = TCH // ROWS_PER_128

_SC_MESH = plsc.VectorSubcoreMesh(core_axis_name="c", subcore_axis_name="s")


def _tr_chunk(src, inb, outb, ia, ibk, n16):
    @plsc.parallel_loop(0, n16, unroll=8)
    def g_body(g):
        ia_g = ia + 2 * g
        for k in range(EMB_K):
            vals = inb[k, pl.ds(g * LANE, LANE)]
            plsc.store_scatter(outb, [ia_g, ibk[k]], vals)


def _tr_body(wt_hbm, ht_hbm, wo_hbm, ho_hbm,
             in0, in1, out0, out1, si0, si1, so0, so1):
    wid = lax.axis_index("s") * NC + lax.axis_index("c")
    base = wid * CPW
    iota = lax.iota(jnp.int32, LANE)
    ia = lax.shift_right_logical(iota, 3)
    ib = lax.shift_left(iota & 7, 4)
    ibk = [ib + k for k in range(EMB_K)]

    for src, dst in ((wt_hbm, wo_hbm), (ht_hbm, ho_hbm)):
        def start_in(c, buf, sem):
            off = pl.multiple_of(c * TCH, TCH)
            pltpu.async_copy(src.at[:, pl.ds(off, TCH)], buf, sem)

        def wait_in(buf, sem):
            pltpu.make_async_copy(src.at[:, pl.ds(0, TCH)], buf, sem).wait()

        def start_out(c, buf, sem):
            pltpu.async_copy(buf, dst.at[pl.ds(c * QCH, QCH)], sem)

        def wait_out(buf, sem):
            pltpu.make_async_copy(buf, dst.at[pl.ds(0, QCH)], sem).wait()

        def process(i, c, bi, bo, sin, sout, bnext, snext):
            @pl.when(i + 1 < CPW)
            def _():
                start_in(c + 1, bnext, snext)
            wait_in(bi, sin)
            @pl.when(i >= 2)
            def _():
                wait_out(bo, sout)
            _tr_chunk(src, bi, bo, ia, ibk, TCH // LANE)
            start_out(c, bo, sout)

        start_in(base, in0, si0)

        def body(i, _):
            c = base + i
            even = (i & 1) == 0

            @pl.when(even)
            def _():
                process(i, c, in0, out0, si0, so0, in1, si1)

            @pl.when(jnp.logical_not(even))
            def _():
                process(i, c, in1, out1, si1, so1, in0, si0)
            return 0

        lax.fori_loop(0, CPW, body, 0)
        wait_out(out0, so0)
        wait_out(out1, so1)

        @pl.when(wid == 30)
        def _():
            start_in(NFULL - 1, in0, si0)
            wait_in(in0, si0)
            _tr_chunk(src, in0, out0, ia, ibk, TCH // LANE)
            start_out(NFULL - 1, out0, so0)
            wait_out(out0, so0)


_sc_transpose = functools.partial(
    pl.kernel,
    mesh=_SC_MESH,
    compiler_params=pltpu.CompilerParams(needs_layout_passes=False),
    out_type=[
        jax.ShapeDtypeStruct((TBL_ROWS, 128), jnp.float32),
        jax.ShapeDtypeStruct((TBL_ROWS, 128), jnp.float32),
    ],
    scratch_types=[
        pltpu.VMEM((EMB_K, TCH), jnp.float32),
        pltpu.VMEM((EMB_K, TCH), jnp.float32),
        pltpu.VMEM((QCH, 128), jnp.float32),
        pltpu.VMEM((QCH, 128), jnp.float32),
        pltpu.SemaphoreType.DMA,
        pltpu.SemaphoreType.DMA,
        pltpu.SemaphoreType.DMA,
        pltpu.SemaphoreType.DMA,
    ],
)(_tr_body)


def _gather_body(uidx_hbm, iidx_hbm, w_hbm, h_hbm, uout_hbm, vout_hbm,
                 idx_v, hi_v, lo_v, rows0, rows1, out16_v, sem0, sem1):
    wid = lax.axis_index("s") * NC + lax.axis_index("c")
    base = wid * BPW
    iota = lax.iota(jnp.int32, LANE)
    zero = iota & 0
    fk = [zero + k for k in range(EMB_K)]
    bufs = (rows0, rows1)
    sems = (sem0, sem1)
    for t in range(2):
        src_idx = uidx_hbm if t == 0 else iidx_hbm
        tbl = w_hbm if t == 0 else h_hbm
        out = uout_hbm if t == 0 else vout_hbm
        pltpu.sync_copy(src_idx.at[pl.ds(base, BPW)], idx_v)
        for i in range(BPW // LANE):
            sl = pl.ds(i * LANE, LANE)
            hi_v[sl] = lax.shift_right_logical(idx_v[sl], 3)
            lo_v[sl] = idx_v[sl] & 7

        outstanding = {}

        def fire(j):
            outstanding[j] = pltpu.async_copy(
                tbl.at[hi_v.at[pl.ds(j * CHUNK, CHUNK)]],
                bufs[j % 2], sems[j % 2],
            )

        fire(0)
        fire(1)
        for j in range(NCHUNK):
            outstanding[j].wait()
            buf = bufs[j % 2]

            @plsc.parallel_loop(0, CHUNK // LANE, unroll=2)
            def cg(g):
                base16 = g * LANE
                idx_a = iota + base16
                s16 = lax.shift_left(lo_v[pl.ds(j * CHUNK + base16, LANE)], 4)
                for k in range(EMB_K):
                    vals = plsc.load_gather(buf, [idx_a, s16 + k])
                    plsc.store_scatter(out16_v, [idx_a + j * CHUNK, fk[k]],
                                       vals)

            if j + 2 < NCHUNK:
                fire(j + 2)

        pltpu.sync_copy(out16_v, out.at[pl.ds(base, BPW)])


_gather = functools.partial(
    pl.kernel,
    mesh=_SC_MESH,
    compiler_params=pltpu.CompilerParams(needs_layout_passes=False),
    out_type=[
        jax.ShapeDtypeStruct((B, EMB_K), jnp.float32),
        jax.ShapeDtypeStruct((B, EMB_K), jnp.float32),
    ],
    scratch_types=[
        pltpu.VMEM((BPW,), jnp.int32),
        pltpu.VMEM((BPW,), jnp.int32),
        pltpu.VMEM((BPW,), jnp.int32),
        pltpu.VMEM((CHUNK, 128), jnp.float32),
        pltpu.VMEM((CHUNK, 128), jnp.float32),
        pltpu.VMEM((BPW, EMB_K), jnp.float32),
        pltpu.SemaphoreType.DMA,
        pltpu.SemaphoreType.DMA,
    ],
)(_gather_body)


BLK = 2048


TAIL0 = NROWS - 64


def _mlp_body(x_ref, u_ref, v_ref, tw_ref, th_ref, w1u_ref, w1v_ref,
              b1_ref, w2t_ref, o_ref):
    iota64 = lax.broadcasted_iota(jnp.int32, (BLK, 64), 1)
    xu = x_ref[...][:, 0:1]
    xi = x_ref[...][:, 1:2]
    oh_u = (xu - TAIL0 == iota64).astype(jnp.float32)
    oh_i = (xi - TAIL0 == iota64).astype(jnp.float32)
    u_eff = jnp.where(
        xu >= TAIL0,
        jnp.dot(oh_u, tw_ref[...], preferred_element_type=jnp.float32),
        u_ref[...],
    )
    v_eff = jnp.where(
        xi >= TAIL0,
        jnp.dot(oh_i, th_ref[...], preferred_element_type=jnp.float32),
        v_ref[...],
    )
    h = (
        jnp.dot(u_eff, w1u_ref[...], preferred_element_type=jnp.float32)
        + jnp.dot(v_eff, w1v_ref[...], preferred_element_type=jnp.float32)
        + b1_ref[...]
    )
    h = jnp.maximum(h, 0.0)
    o_ref[...] = jnp.dot(h, w2t_ref[...], preferred_element_type=jnp.float32)


def _mlp(x, u16, v16, tw, th, w1u, w1v, b1_2d, w2t):
    grid = B // BLK
    return pl.pallas_call(
        _mlp_body,
        grid=(grid,),
        in_specs=[
            pl.BlockSpec((BLK, 2), lambda i: (i, 0)),
            pl.BlockSpec((BLK, EMB_K), lambda i: (i, 0)),
            pl.BlockSpec((BLK, EMB_K), lambda i: (i, 0)),
            pl.BlockSpec((64, EMB_K), lambda i: (0, 0)),
            pl.BlockSpec((64, EMB_K), lambda i: (0, 0)),
            pl.BlockSpec((EMB_K, EMB_K), lambda i: (0, 0)),
            pl.BlockSpec((EMB_K, EMB_K), lambda i: (0, 0)),
            pl.BlockSpec((1, EMB_K), lambda i: (0, 0)),
            pl.BlockSpec((EMB_K, 1), lambda i: (0, 0)),
        ],
        out_specs=pl.BlockSpec((BLK, 1), lambda i: (i, 0)),
        out_shape=jax.ShapeDtypeStruct((B, 1), jnp.float32),
    )(x, u16, v16, tw, th, w1u, w1v, b1_2d, w2t)


@jax.jit
def kernel(x, W, H, W1, b1, W2):
    user_idx = x[:, 0]
    item_idx = x[:, 1]
    W128, H128 = _sc_transpose(W.T, H.T)
    U16, V16 = _gather(user_idx, item_idx, W128, H128)
    tw = W[TAIL0:, :]
    th = H[TAIL0:, :]
    w1u = W1[:, :EMB_K].T
    w1v = W1[:, EMB_K:].T
    return _mlp(x, U16, V16, tw, th, w1u, w1v,
                b1.reshape(1, EMB_K), W2.T)


# speedup vs baseline: 1.3157x; 1.3157x over previous
import functools

import jax
import jax.numpy as jnp
import numpy as np
from jax import lax
from jax.experimental import pallas as pl
from jax.experimental.pallas import tpu as pltpu
from jax.experimental.pallas import tpu_sc as plsc

B = 16384
EMB_K = 16
NROWS = 1000000
ROWS_PER_128 = 8
TBL_ROWS = NROWS // ROWS_PER_128
NC = 2
NS = 16
NW = NC * NS
BPW = B // NW
CHUNK = 128
NCHUNK = BPW // CHUNK
LANE = 16

TCH = 512
NFULL = NROWS // TCH
CPW = NFULL // NW
QCH = TCH // ROWS_PER_128

_SC_MESH = plsc.VectorSubcoreMesh(core_axis_name="c", subcore_axis_name="s")


def _tr_chunk(src, inb, outb, ia, ibk, n16):
    @plsc.parallel_loop(0, n16, unroll=2)
    def g_body(g):
        ia_g = ia + 2 * g
        for k in range(EMB_K):
            vals = inb[k, pl.ds(g * LANE, LANE)]
            plsc.store_scatter(outb, [ia_g, ibk[k]], vals)


def _tr_body(wt_hbm, ht_hbm, wo_hbm, ho_hbm,
             in0, in1, out0, out1, si0, si1, so0, so1):
    wid = lax.axis_index("s") * NC + lax.axis_index("c")
    base = wid * CPW
    iota = lax.iota(jnp.int32, LANE)
    ia = lax.shift_right_logical(iota, 3)
    ib = lax.shift_left(iota & 7, 4)
    ibk = [ib + k for k in range(EMB_K)]

    for src, dst in ((wt_hbm, wo_hbm), (ht_hbm, ho_hbm)):
        def start_in(c, buf, sem):
            off = pl.multiple_of(c * TCH, TCH)
            pltpu.async_copy(src.at[:, pl.ds(off, TCH)], buf, sem)

        def wait_in(buf, sem):
            pltpu.make_async_copy(src.at[:, pl.ds(0, TCH)], buf, sem).wait()

        def start_out(c, buf, sem):
            pltpu.async_copy(buf, dst.at[pl.ds(c * QCH, QCH)], sem)

        def wait_out(buf, sem):
            pltpu.make_async_copy(buf, dst.at[pl.ds(0, QCH)], sem).wait()

        def process(i, c, bi, bo, sin, sout, bnext, snext):
            @pl.when(i + 1 < CPW)
            def _():
                start_in(c + 1, bnext, snext)
            wait_in(bi, sin)
            @pl.when(i >= 2)
            def _():
                wait_out(bo, sout)
            _tr_chunk(src, bi, bo, ia, ibk, TCH // LANE)
            start_out(c, bo, sout)

        start_in(base, in0, si0)

        def body(i, _):
            c = base + i
            even = (i & 1) == 0

            @pl.when(even)
            def _():
                process(i, c, in0, out0, si0, so0, in1, si1)

            @pl.when(jnp.logical_not(even))
            def _():
                process(i, c, in1, out1, si1, so1, in0, si0)
            return 0

        lax.fori_loop(0, CPW, body, 0)
        wait_out(out0, so0)
        wait_out(out1, so1)

        @pl.when(wid == 30)
        def _():
            start_in(NFULL - 1, in0, si0)
            wait_in(in0, si0)
            _tr_chunk(src, in0, out0, ia, ibk, TCH // LANE)
            start_out(NFULL - 1, out0, so0)
            wait_out(out0, so0)


_sc_transpose = functools.partial(
    pl.kernel,
    mesh=_SC_MESH,
    compiler_params=pltpu.CompilerParams(needs_layout_passes=False),
    out_type=[
        jax.ShapeDtypeStruct((TBL_ROWS, 128), jnp.float32),
        jax.ShapeDtypeStruct((TBL_ROWS, 128), jnp.float32),
    ],
    scratch_types=[
        pltpu.VMEM((EMB_K, TCH), jnp.float32),
        pltpu.VMEM((EMB_K, TCH), jnp.float32),
        pltpu.VMEM((QCH, 128), jnp.float32),
        pltpu.VMEM((QCH, 128), jnp.float32),
        pltpu.SemaphoreType.DMA,
        pltpu.SemaphoreType.DMA,
        pltpu.SemaphoreType.DMA,
        pltpu.SemaphoreType.DMA,
    ],
)(_tr_body)


def _gather_body(uidx_hbm, iidx_hbm, w_hbm, h_hbm, uout_hbm, vout_hbm,
                 idx_v, hi_v, lo_v, rows0, rows1, out16_v, sem0, sem1):
    wid = lax.axis_index("s") * NC + lax.axis_index("c")
    base = wid * BPW
    iota = lax.iota(jnp.int32, LANE)
    zero = iota & 0
    fk = [zero + k for k in range(EMB_K)]
    bufs = (rows0, rows1)
    sems = (sem0, sem1)
    for t in range(2):
        src_idx = uidx_hbm if t == 0 else iidx_hbm
        tbl = w_hbm if t == 0 else h_hbm
        out = uout_hbm if t == 0 else vout_hbm
        pltpu.sync_copy(src_idx.at[pl.ds(base, BPW)], idx_v)
        for i in range(BPW // LANE):
            sl = pl.ds(i * LANE, LANE)
            hi_v[sl] = lax.shift_right_logical(idx_v[sl], 3)
            lo_v[sl] = idx_v[sl] & 7

        outstanding = {}

        def fire(j):
            outstanding[j] = pltpu.async_copy(
                tbl.at[hi_v.at[pl.ds(j * CHUNK, CHUNK)]],
                bufs[j % 2], sems[j % 2],
            )

        fire(0)
        fire(1)
        for j in range(NCHUNK):
            outstanding[j].wait()
            buf = bufs[j % 2]

            @plsc.parallel_loop(0, CHUNK // LANE, unroll=2)
            def cg(g):
                base16 = g * LANE
                idx_a = iota + base16
                s16 = lax.shift_left(lo_v[pl.ds(j * CHUNK + base16, LANE)], 4)
                for k in range(EMB_K):
                    vals = plsc.load_gather(buf, [idx_a, s16 + k])
                    plsc.store_scatter(out16_v, [idx_a + j * CHUNK, fk[k]],
                                       vals)

            if j + 2 < NCHUNK:
                fire(j + 2)

        pltpu.sync_copy(out16_v, out.at[pl.ds(base, BPW)])


_gather = functools.partial(
    pl.kernel,
    mesh=_SC_MESH,
    compiler_params=pltpu.CompilerParams(needs_layout_passes=False),
    out_type=[
        jax.ShapeDtypeStruct((B, EMB_K), jnp.float32),
        jax.ShapeDtypeStruct((B, EMB_K), jnp.float32),
    ],
    scratch_types=[
        pltpu.VMEM((BPW,), jnp.int32),
        pltpu.VMEM((BPW,), jnp.int32),
        pltpu.VMEM((BPW,), jnp.int32),
        pltpu.VMEM((CHUNK, 128), jnp.float32),
        pltpu.VMEM((CHUNK, 128), jnp.float32),
        pltpu.VMEM((BPW, EMB_K), jnp.float32),
        pltpu.SemaphoreType.DMA,
        pltpu.SemaphoreType.DMA,
    ],
)(_gather_body)


BLK = 2048


TAIL0 = NROWS - 64


def _mlp_body(x_ref, u_ref, v_ref, tw_ref, th_ref, w1u_ref, w1v_ref,
              b1_ref, w2t_ref, o_ref):
    iota64 = lax.broadcasted_iota(jnp.int32, (BLK, 64), 1)
    xu = x_ref[...][:, 0:1]
    xi = x_ref[...][:, 1:2]
    oh_u = (xu - TAIL0 == iota64).astype(jnp.float32)
    oh_i = (xi - TAIL0 == iota64).astype(jnp.float32)
    u_eff = jnp.where(
        xu >= TAIL0,
        jnp.dot(oh_u, tw_ref[...], preferred_element_type=jnp.float32),
        u_ref[...],
    )
    v_eff = jnp.where(
        xi >= TAIL0,
        jnp.dot(oh_i, th_ref[...], preferred_element_type=jnp.float32),
        v_ref[...],
    )
    h = (
        jnp.dot(u_eff, w1u_ref[...], preferred_element_type=jnp.float32)
        + jnp.dot(v_eff, w1v_ref[...], preferred_element_type=jnp.float32)
        + b1_ref[...]
    )
    h = jnp.maximum(h, 0.0)
    o_ref[...] = jnp.dot(h, w2t_ref[...], preferred_element_type=jnp.float32)


def _mlp(x, u16, v16, tw, th, w1u, w1v, b1_2d, w2t):
    grid = B // BLK
    return pl.pallas_call(
        _mlp_body,
        grid=(grid,),
        in_specs=[
            pl.BlockSpec((BLK, 2), lambda i: (i, 0)),
            pl.BlockSpec((BLK, EMB_K), lambda i: (i, 0)),
            pl.BlockSpec((BLK, EMB_K), lambda i: (i, 0)),
            pl.BlockSpec((64, EMB_K), lambda i: (0, 0)),
            pl.BlockSpec((64, EMB_K), lambda i: (0, 0)),
            pl.BlockSpec((EMB_K, EMB_K), lambda i: (0, 0)),
            pl.BlockSpec((EMB_K, EMB_K), lambda i: (0, 0)),
            pl.BlockSpec((1, EMB_K), lambda i: (0, 0)),
            pl.BlockSpec((EMB_K, 1), lambda i: (0, 0)),
        ],
        out_specs=pl.BlockSpec((BLK, 1), lambda i: (i, 0)),
        out_shape=jax.ShapeDtypeStruct((B, 1), jnp.float32),
    )(x, u16, v16, tw, th, w1u, w1v, b1_2d, w2t)


@jax.jit
def kernel(x, W, H, W1, b1, W2):
    user_idx = x[:, 0]
    item_idx = x[:, 1]
    W128, H128 = _sc_transpose(W.T, H.T)
    U16, V16 = _gather(user_idx, item_idx, W128, H128)
    tw = W[TAIL0:, :]
    th = H[TAIL0:, :]
    w1u = W1[:, :EMB_K].T
    w1v = W1[:, EMB_K:].T
    return _mlp(x, U16, V16, tw, th, w1u, w1v,
                b1.reshape(1, EMB_K), W2.T)


# speedup vs baseline: 1.5870x; 1.2062x over previous
import functools

import jax
import jax.numpy as jnp
import numpy as np
from jax import lax
from jax.experimental import pallas as pl
from jax.experimental.pallas import tpu as pltpu
from jax.experimental.pallas import tpu_sc as plsc

B = 16384
EMB_K = 16
NROWS = 1000000
ROWS_PER_128 = 8
TBL_ROWS = NROWS // ROWS_PER_128
NC = 2
NS = 16
NW = NC * NS
BPW = B // NW
CHUNK = 128
NCHUNK = BPW // CHUNK
LANE = 16

TCH = 512
NFULL = NROWS // TCH
CPW = NFULL // NW
QCH = TCH // ROWS_PER_128

_SC_MESH = plsc.VectorSubcoreMesh(core_axis_name="c", subcore_axis_name="s")


def _tr_chunk(src, inb, outb, ia, ibk, n16):
    @plsc.parallel_loop(0, n16, unroll=1)
    def g_body(g):
        ia_g = ia + 2 * g
        for k in range(EMB_K):
            vals = inb[k, pl.ds(g * LANE, LANE)]
            plsc.store_scatter(outb, [ia_g, ibk[k]], vals)


def _tr_body(wt_hbm, ht_hbm, wo_hbm, ho_hbm,
             in0, in1, out0, out1, si0, si1, so0, so1):
    wid = lax.axis_index("s") * NC + lax.axis_index("c")
    base = wid * CPW
    iota = lax.iota(jnp.int32, LANE)
    ia = lax.shift_right_logical(iota, 3)
    ib = lax.shift_left(iota & 7, 4)
    ibk = [ib + k for k in range(EMB_K)]

    for src, dst in ((wt_hbm, wo_hbm), (ht_hbm, ho_hbm)):
        def start_in(c, buf, sem):
            off = pl.multiple_of(c * TCH, TCH)
            pltpu.async_copy(src.at[:, pl.ds(off, TCH)], buf, sem)

        def wait_in(buf, sem):
            pltpu.make_async_copy(src.at[:, pl.ds(0, TCH)], buf, sem).wait()

        def start_out(c, buf, sem):
            pltpu.async_copy(buf, dst.at[pl.ds(c * QCH, QCH)], sem)

        def wait_out(buf, sem):
            pltpu.make_async_copy(buf, dst.at[pl.ds(0, QCH)], sem).wait()

        def process(i, c, bi, bo, sin, sout, bnext, snext):
            @pl.when(i + 1 < CPW)
            def _():
                start_in(c + 1, bnext, snext)
            wait_in(bi, sin)
            @pl.when(i >= 2)
            def _():
                wait_out(bo, sout)
            _tr_chunk(src, bi, bo, ia, ibk, TCH // LANE)
            start_out(c, bo, sout)

        start_in(base, in0, si0)

        def body(i, _):
            c = base + i
            even = (i & 1) == 0

            @pl.when(even)
            def _():
                process(i, c, in0, out0, si0, so0, in1, si1)

            @pl.when(jnp.logical_not(even))
            def _():
                process(i, c, in1, out1, si1, so1, in0, si0)
            return 0

        lax.fori_loop(0, CPW, body, 0)
        wait_out(out0, so0)
        wait_out(out1, so1)

        @pl.when(wid == 30)
        def _():
            start_in(NFULL - 1, in0, si0)
            wait_in(in0, si0)
            _tr_chunk(src, in0, out0, ia, ibk, TCH // LANE)
            start_out(NFULL - 1, out0, so0)
            wait_out(out0, so0)


_sc_transpose = functools.partial(
    pl.kernel,
    mesh=_SC_MESH,
    compiler_params=pltpu.CompilerParams(needs_layout_passes=False),
    out_type=[
        jax.ShapeDtypeStruct((TBL_ROWS, 128), jnp.float32),
        jax.ShapeDtypeStruct((TBL_ROWS, 128), jnp.float32),
    ],
    scratch_types=[
        pltpu.VMEM((EMB_K, TCH), jnp.float32),
        pltpu.VMEM((EMB_K, TCH), jnp.float32),
        pltpu.VMEM((QCH, 128), jnp.float32),
        pltpu.VMEM((QCH, 128), jnp.float32),
        pltpu.SemaphoreType.DMA,
        pltpu.SemaphoreType.DMA,
        pltpu.SemaphoreType.DMA,
        pltpu.SemaphoreType.DMA,
    ],
)(_tr_body)


def _gather_body(uidx_hbm, iidx_hbm, w_hbm, h_hbm, uout_hbm, vout_hbm,
                 idx_v, hi_v, lo_v, rows0, rows1, out16_v, sem0, sem1):
    wid = lax.axis_index("s") * NC + lax.axis_index("c")
    base = wid * BPW
    iota = lax.iota(jnp.int32, LANE)
    zero = iota & 0
    fk = [zero + k for k in range(EMB_K)]
    bufs = (rows0, rows1)
    sems = (sem0, sem1)
    for t in range(2):
        src_idx = uidx_hbm if t == 0 else iidx_hbm
        tbl = w_hbm if t == 0 else h_hbm
        out = uout_hbm if t == 0 else vout_hbm
        pltpu.sync_copy(src_idx.at[pl.ds(base, BPW)], idx_v)
        for i in range(BPW // LANE):
            sl = pl.ds(i * LANE, LANE)
            hi_v[sl] = lax.shift_right_logical(idx_v[sl], 3)
            lo_v[sl] = idx_v[sl] & 7

        outstanding = {}

        def fire(j):
            outstanding[j] = pltpu.async_copy(
                tbl.at[hi_v.at[pl.ds(j * CHUNK, CHUNK)]],
                bufs[j % 2], sems[j % 2],
            )

        fire(0)
        fire(1)
        for j in range(NCHUNK):
            outstanding[j].wait()
            buf = bufs[j % 2]

            @plsc.parallel_loop(0, CHUNK // LANE, unroll=2)
            def cg(g):
                base16 = g * LANE
                idx_a = iota + base16
                s16 = lax.shift_left(lo_v[pl.ds(j * CHUNK + base16, LANE)], 4)
                for k in range(EMB_K):
                    vals = plsc.load_gather(buf, [idx_a, s16 + k])
                    plsc.store_scatter(out16_v, [idx_a + j * CHUNK, fk[k]],
                                       vals)

            if j + 2 < NCHUNK:
                fire(j + 2)

        pltpu.sync_copy(out16_v, out.at[pl.ds(base, BPW)])


_gather = functools.partial(
    pl.kernel,
    mesh=_SC_MESH,
    compiler_params=pltpu.CompilerParams(needs_layout_passes=False),
    out_type=[
        jax.ShapeDtypeStruct((B, EMB_K), jnp.float32),
        jax.ShapeDtypeStruct((B, EMB_K), jnp.float32),
    ],
    scratch_types=[
        pltpu.VMEM((BPW,), jnp.int32),
        pltpu.VMEM((BPW,), jnp.int32),
        pltpu.VMEM((BPW,), jnp.int32),
        pltpu.VMEM((CHUNK, 128), jnp.float32),
        pltpu.VMEM((CHUNK, 128), jnp.float32),
        pltpu.VMEM((BPW, EMB_K), jnp.float32),
        pltpu.SemaphoreType.DMA,
        pltpu.SemaphoreType.DMA,
    ],
)(_gather_body)


BLK = 2048


TAIL0 = NROWS - 64


def _mlp_body(x_ref, u_ref, v_ref, tw_ref, th_ref, w1u_ref, w1v_ref,
              b1_ref, w2t_ref, o_ref):
    iota64 = lax.broadcasted_iota(jnp.int32, (BLK, 64), 1)
    xu = x_ref[...][:, 0:1]
    xi = x_ref[...][:, 1:2]
    oh_u = (xu - TAIL0 == iota64).astype(jnp.float32)
    oh_i = (xi - TAIL0 == iota64).astype(jnp.float32)
    u_eff = jnp.where(
        xu >= TAIL0,
        jnp.dot(oh_u, tw_ref[...], preferred_element_type=jnp.float32),
        u_ref[...],
    )
    v_eff = jnp.where(
        xi >= TAIL0,
        jnp.dot(oh_i, th_ref[...], preferred_element_type=jnp.float32),
        v_ref[...],
    )
    h = (
        jnp.dot(u_eff, w1u_ref[...], preferred_element_type=jnp.float32)
        + jnp.dot(v_eff, w1v_ref[...], preferred_element_type=jnp.float32)
        + b1_ref[...]
    )
    h = jnp.maximum(h, 0.0)
    o_ref[...] = jnp.dot(h, w2t_ref[...], preferred_element_type=jnp.float32)


def _mlp(x, u16, v16, tw, th, w1u, w1v, b1_2d, w2t):
    grid = B // BLK
    return pl.pallas_call(
        _mlp_body,
        grid=(grid,),
        in_specs=[
            pl.BlockSpec((BLK, 2), lambda i: (i, 0)),
            pl.BlockSpec((BLK, EMB_K), lambda i: (i, 0)),
            pl.BlockSpec((BLK, EMB_K), lambda i: (i, 0)),
            pl.BlockSpec((64, EMB_K), lambda i: (0, 0)),
            pl.BlockSpec((64, EMB_K), lambda i: (0, 0)),
            pl.BlockSpec((EMB_K, EMB_K), lambda i: (0, 0)),
            pl.BlockSpec((EMB_K, EMB_K), lambda i: (0, 0)),
            pl.BlockSpec((1, EMB_K), lambda i: (0, 0)),
            pl.BlockSpec((EMB_K, 1), lambda i: (0, 0)),
        ],
        out_specs=pl.BlockSpec((BLK, 1), lambda i: (i, 0)),
        out_shape=jax.ShapeDtypeStruct((B, 1), jnp.float32),
    )(x, u16, v16, tw, th, w1u, w1v, b1_2d, w2t)


@jax.jit
def kernel(x, W, H, W1, b1, W2):
    user_idx = x[:, 0]
    item_idx = x[:, 1]
    W128, H128 = _sc_transpose(W.T, H.T)
    U16, V16 = _gather(user_idx, item_idx, W128, H128)
    tw = W[TAIL0:, :]
    th = H[TAIL0:, :]
    w1u = W1[:, :EMB_K].T
    w1v = W1[:, EMB_K:].T
    return _mlp(x, U16, V16, tw, th, w1u, w1v,
                b1.reshape(1, EMB_K), W2.T)


# speedup vs baseline: 1.6019x; 1.0094x over previous
import functools

import jax
import jax.numpy as jnp
import numpy as np
from jax import lax
from jax.experimental import pallas as pl
from jax.experimental.pallas import tpu as pltpu
from jax.experimental.pallas import tpu_sc as plsc

B = 16384
EMB_K = 16
NROWS = 1000000
ROWS_PER_128 = 8
TBL_ROWS = NROWS // ROWS_PER_128
NC = 2
NS = 16
NW = NC * NS
BPW = B // NW
CHUNK = 128
NCHUNK = BPW // CHUNK
LANE = 16

TCH = 512
NFULL = NROWS // TCH
CPW = NFULL // NW
QCH = TCH // ROWS_PER_128

_SC_MESH = plsc.VectorSubcoreMesh(core_axis_name="c", subcore_axis_name="s")


def _tr_chunk(src, inb, outb, ia, ibk, n16):
    @plsc.parallel_loop(0, n16, unroll=1)
    def g_body(g):
        ia_g = ia + 2 * g
        for k in range(EMB_K):
            vals = inb[k, pl.ds(g * LANE, LANE)]
            plsc.store_scatter(outb, [ia_g, ibk[k]], vals)


def _tr_body(wt_hbm, ht_hbm, wo_hbm, ho_hbm,
             in0, in1, out0, out1, si0, si1, so0, so1):
    wid = lax.axis_index("s") * NC + lax.axis_index("c")
    base = wid * CPW
    iota = lax.iota(jnp.int32, LANE)
    ia = lax.shift_right_logical(iota, 3)
    ib = lax.shift_left(iota & 7, 4)
    ibk = [ib + k for k in range(EMB_K)]

    for src, dst in ((wt_hbm, wo_hbm), (ht_hbm, ho_hbm)):
        def start_in(c, buf, sem):
            off = pl.multiple_of(c * TCH, TCH)
            pltpu.async_copy(src.at[:, pl.ds(off, TCH)], buf, sem)

        def wait_in(buf, sem):
            pltpu.make_async_copy(src.at[:, pl.ds(0, TCH)], buf, sem).wait()

        def start_out(c, buf, sem):
            pltpu.async_copy(buf, dst.at[pl.ds(c * QCH, QCH)], sem)

        def wait_out(buf, sem):
            pltpu.make_async_copy(buf, dst.at[pl.ds(0, QCH)], sem).wait()

        def process(i, c, bi, bo, sin, sout, bnext, snext):
            @pl.when(i + 1 < CPW)
            def _():
                start_in(c + 1, bnext, snext)
            wait_in(bi, sin)
            @pl.when(i >= 2)
            def _():
                wait_out(bo, sout)
            _tr_chunk(src, bi, bo, ia, ibk, TCH // LANE)
            start_out(c, bo, sout)

        start_in(base, in0, si0)

        def body(i, _):
            c = base + i
            even = (i & 1) == 0

            @pl.when(even)
            def _():
                process(i, c, in0, out0, si0, so0, in1, si1)

            @pl.when(jnp.logical_not(even))
            def _():
                process(i, c, in1, out1, si1, so1, in0, si0)
            return 0

        lax.fori_loop(0, CPW, body, 0)
        wait_out(out0, so0)
        wait_out(out1, so1)

        @pl.when(wid == 30)
        def _():
            start_in(NFULL - 1, in0, si0)
            wait_in(in0, si0)
            _tr_chunk(src, in0, out0, ia, ibk, TCH // LANE)
            start_out(NFULL - 1, out0, so0)
            wait_out(out0, so0)


_sc_transpose = functools.partial(
    pl.kernel,
    mesh=_SC_MESH,
    compiler_params=pltpu.CompilerParams(needs_layout_passes=False),
    out_type=[
        jax.ShapeDtypeStruct((TBL_ROWS, 128), jnp.float32),
        jax.ShapeDtypeStruct((TBL_ROWS, 128), jnp.float32),
    ],
    scratch_types=[
        pltpu.VMEM((EMB_K, TCH), jnp.float32),
        pltpu.VMEM((EMB_K, TCH), jnp.float32),
        pltpu.VMEM((QCH, 128), jnp.float32),
        pltpu.VMEM((QCH, 128), jnp.float32),
        pltpu.SemaphoreType.DMA,
        pltpu.SemaphoreType.DMA,
        pltpu.SemaphoreType.DMA,
        pltpu.SemaphoreType.DMA,
    ],
)(_tr_body)


def _gather_body(uidx_hbm, iidx_hbm, w_hbm, h_hbm, uout_hbm, vout_hbm,
                 idx_v, hi_v, lo_v, rows0, rows1, out16_v, sem0, sem1):
    wid = lax.axis_index("s") * NC + lax.axis_index("c")
    base = wid * BPW
    iota = lax.iota(jnp.int32, LANE)
    zero = iota & 0
    fk = [zero + k for k in range(EMB_K)]
    bufs = (rows0, rows1)
    sems = (sem0, sem1)
    for t in range(2):
        src_idx = uidx_hbm if t == 0 else iidx_hbm
        tbl = w_hbm if t == 0 else h_hbm
        out = uout_hbm if t == 0 else vout_hbm
        pltpu.sync_copy(src_idx.at[pl.ds(base, BPW)], idx_v)
        for i in range(BPW // LANE):
            sl = pl.ds(i * LANE, LANE)
            hi_v[sl] = lax.shift_right_logical(idx_v[sl], 3)
            lo_v[sl] = idx_v[sl] & 7

        outstanding = {}

        def fire(j):
            outstanding[j] = pltpu.async_copy(
                tbl.at[hi_v.at[pl.ds(j * CHUNK, CHUNK)]],
                bufs[j % 2], sems[j % 2],
            )

        fire(0)
        fire(1)
        for j in range(NCHUNK):
            outstanding[j].wait()
            buf = bufs[j % 2]

            @plsc.parallel_loop(0, CHUNK // LANE, unroll=1)
            def cg(g):
                base16 = g * LANE
                idx_a = iota + base16
                s16 = lax.shift_left(lo_v[pl.ds(j * CHUNK + base16, LANE)], 4)
                for k in range(EMB_K):
                    vals = plsc.load_gather(buf, [idx_a, s16 + k])
                    plsc.store_scatter(out16_v, [idx_a + j * CHUNK, fk[k]],
                                       vals)

            if j + 2 < NCHUNK:
                fire(j + 2)

        pltpu.sync_copy(out16_v, out.at[pl.ds(base, BPW)])


_gather = functools.partial(
    pl.kernel,
    mesh=_SC_MESH,
    compiler_params=pltpu.CompilerParams(needs_layout_passes=False),
    out_type=[
        jax.ShapeDtypeStruct((B, EMB_K), jnp.float32),
        jax.ShapeDtypeStruct((B, EMB_K), jnp.float32),
    ],
    scratch_types=[
        pltpu.VMEM((BPW,), jnp.int32),
        pltpu.VMEM((BPW,), jnp.int32),
        pltpu.VMEM((BPW,), jnp.int32),
        pltpu.VMEM((CHUNK, 128), jnp.float32),
        pltpu.VMEM((CHUNK, 128), jnp.float32),
        pltpu.VMEM((BPW, EMB_K), jnp.float32),
        pltpu.SemaphoreType.DMA,
        pltpu.SemaphoreType.DMA,
    ],
)(_gather_body)


BLK = 2048


TAIL0 = NROWS - 64


def _mlp_body(x_ref, u_ref, v_ref, tw_ref, th_ref, w1u_ref, w1v_ref,
              b1_ref, w2t_ref, o_ref):
    iota64 = lax.broadcasted_iota(jnp.int32, (BLK, 64), 1)
    xu = x_ref[...][:, 0:1]
    xi = x_ref[...][:, 1:2]
    oh_u = (xu - TAIL0 == iota64).astype(jnp.float32)
    oh_i = (xi - TAIL0 == iota64).astype(jnp.float32)
    u_eff = jnp.where(
        xu >= TAIL0,
        jnp.dot(oh_u, tw_ref[...], preferred_element_type=jnp.float32),
        u_ref[...],
    )
    v_eff = jnp.where(
        xi >= TAIL0,
        jnp.dot(oh_i, th_ref[...], preferred_element_type=jnp.float32),
        v_ref[...],
    )
    h = (
        jnp.dot(u_eff, w1u_ref[...], preferred_element_type=jnp.float32)
        + jnp.dot(v_eff, w1v_ref[...], preferred_element_type=jnp.float32)
        + b1_ref[...]
    )
    h = jnp.maximum(h, 0.0)
    o_ref[...] = jnp.dot(h, w2t_ref[...], preferred_element_type=jnp.float32)


def _mlp(x, u16, v16, tw, th, w1u, w1v, b1_2d, w2t):
    grid = B // BLK
    return pl.pallas_call(
        _mlp_body,
        grid=(grid,),
        in_specs=[
            pl.BlockSpec((BLK, 2), lambda i: (i, 0)),
            pl.BlockSpec((BLK, EMB_K), lambda i: (i, 0)),
            pl.BlockSpec((BLK, EMB_K), lambda i: (i, 0)),
            pl.BlockSpec((64, EMB_K), lambda i: (0, 0)),
            pl.BlockSpec((64, EMB_K), lambda i: (0, 0)),
            pl.BlockSpec((EMB_K, EMB_K), lambda i: (0, 0)),
            pl.BlockSpec((EMB_K, EMB_K), lambda i: (0, 0)),
            pl.BlockSpec((1, EMB_K), lambda i: (0, 0)),
            pl.BlockSpec((EMB_K, 1), lambda i: (0, 0)),
        ],
        out_specs=pl.BlockSpec((BLK, 1), lambda i: (i, 0)),
        out_shape=jax.ShapeDtypeStruct((B, 1), jnp.float32),
    )(x, u16, v16, tw, th, w1u, w1v, b1_2d, w2t)


@jax.jit
def kernel(x, W, H, W1, b1, W2):
    user_idx = x[:, 0]
    item_idx = x[:, 1]
    W128, H128 = _sc_transpose(W.T, H.T)
    U16, V16 = _gather(user_idx, item_idx, W128, H128)
    tw = W[TAIL0:, :]
    th = H[TAIL0:, :]
    w1u = W1[:, :EMB_K].T
    w1v = W1[:, EMB_K:].T
    return _mlp(x, U16, V16, tw, th, w1u, w1v,
                b1.reshape(1, EMB_K), W2.T)
